# baseline (speedup 1.0000x reference)
import functools
import math

import jax
import jax.numpy as jnp
import numpy as np
from jax.experimental import pallas as pl
from jax.experimental.pallas import tpu as pltpu

N = 4096
H = 64
W = 64
HW = H * W
TANX = 0.5
TANY = 0.5
FX = W / (2.0 * TANX)
FY = H / (2.0 * TANY)
LIMX = 1.3 * TANX
LIMY = 1.3 * TANY
LOG_EPS = math.log(1e-4)

P_BLK = 1024
K_BLK = 256
NC = N // K_BLK
PB = HW // P_BLK

_yy, _xx = np.mgrid[0:H, 0:W]
_r2 = (_xx - (W - 1) / 2.0) ** 2 + (_yy - (H - 1) / 2.0) ** 2
_PERM = np.argsort(_r2.reshape(-1), kind="stable").astype(np.int32)
_INV_PERM = np.argsort(_PERM, kind="stable").astype(np.int32)
_PG = np.stack([(_PERM % W).astype(np.float32),
                (_PERM // W).astype(np.float32)], axis=1)
_R2S = np.sort(_r2.reshape(-1), kind="stable")
_BLK_R0 = [float(np.sqrt(_R2S[b * P_BLK])) for b in range(PB)]
_BLK_R1 = [float(np.sqrt(_R2S[(b + 1) * P_BLK - 1])) for b in range(PB)]
_CX = (W - 1) / 2.0
_CY = (H - 1) / 2.0


def _bf(x):
    return x.astype(jnp.bfloat16).astype(jnp.float32)


def _project_kernel(mt_ref, st_ref, qt_ref, ot_ref, ct_ref, vm_ref, pm_ref,
                    params_ref, colinv_ref, radii_ref):
    f32 = jnp.float32
    mx = mt_ref[0:1, :]
    my = mt_ref[1:2, :]
    mz = mt_ref[2:3, :]
    s = [st_ref[i:i + 1, :] for i in range(3)]
    qr = qt_ref[0:1, :]
    qx = qt_ref[1:2, :]
    qy = qt_ref[2:3, :]
    qz = qt_ref[3:4, :]
    nrm = jnp.sqrt(qr * qr + qx * qx + qy * qy + qz * qz) + 1e-12
    r = qr / nrm
    x = qx / nrm
    y = qy / nrm
    z = qz / nrm
    R = [
        [1 - 2 * (y * y + z * z), 2 * (x * y - r * z), 2 * (x * z + r * y)],
        [2 * (x * y + r * z), 1 - 2 * (x * x + z * z), 2 * (y * z - r * x)],
        [2 * (x * z - r * y), 2 * (y * z + r * x), 1 - 2 * (x * x + y * y)],
    ]
    M = [[_bf(R[a][j] * s[j]) for j in range(3)] for a in range(3)]
    cov3 = [[sum(M[a][j] * M[b][j] for j in range(3)) for b in range(3)]
            for a in range(3)]

    vm = [[vm_ref[i, j] for j in range(4)] for i in range(4)]
    pm = [[pm_ref[i, j] for j in range(4)] for i in range(4)]
    mxb, myb, mzb = _bf(mx), _bf(my), _bf(mz)
    tx = vm[0][0] * mxb + vm[0][1] * myb + vm[0][2] * mzb + vm[0][3]
    ty = vm[1][0] * mxb + vm[1][1] * myb + vm[1][2] * mzb + vm[1][3]
    tz = vm[2][0] * mxb + vm[2][1] * myb + vm[2][2] * mzb + vm[2][3]
    ph0 = pm[0][0] * mxb + pm[0][1] * myb + pm[0][2] * mzb + pm[0][3]
    ph1 = pm[1][0] * mxb + pm[1][1] * myb + pm[1][2] * mzb + pm[1][3]
    ph3 = pm[3][0] * mxb + pm[3][1] * myb + pm[3][2] * mzb + pm[3][3]
    pw = 1.0 / (ph3 + 1e-7)
    ppx = ph0 * pw
    ppy = ph1 * pw

    tzc = jnp.where(jnp.abs(tz) < 1e-6, 1e-6, tz)
    txc = jnp.clip(tx / tzc, -LIMX, LIMX) * tzc
    tyc = jnp.clip(ty / tzc, -LIMY, LIMY) * tzc
    itz = 1.0 / tzc
    J0 = [_bf(FX / tzc), jnp.zeros_like(itz), _bf(-FX * txc / (tzc * tzc))]
    J1 = [jnp.zeros_like(itz), _bf(FY / tzc), _bf(-FY * tyc / (tzc * tzc))]
    Tm0 = [J0[0] * vm[0][k] + J0[2] * vm[2][k] for k in range(3)]
    Tm1 = [J1[1] * vm[1][k] + J1[2] * vm[2][k] for k in range(3)]
    Tm0b = [_bf(t) for t in Tm0]
    Tm1b = [_bf(t) for t in Tm1]
    cov3b = [[_bf(cov3[a][b]) for b in range(3)] for a in range(3)]
    u0 = [_bf(sum(Tm0b[k] * cov3b[k][j] for k in range(3))) for j in range(3)]
    u1 = [_bf(sum(Tm1b[k] * cov3b[k][j] for k in range(3))) for j in range(3)]
    cov00 = sum(u0[j] * Tm0b[j] for j in range(3))
    cov01 = sum(u0[j] * Tm1b[j] for j in range(3))
    cov11 = sum(u1[j] * Tm1b[j] for j in range(3))

    a = cov00 + 0.3
    b = cov01
    c = cov11 + 0.3
    det = a * c - b * b
    valid = (det > 0.0) & (tz > 0.2)
    det_safe = jnp.where(valid, det, 1.0)
    conA = c / det_safe
    conB = -b / det_safe
    conC = a / det_safe
    px = ((ppx + 1.0) * W - 1.0) * 0.5
    py = ((ppy + 1.0) * H - 1.0) * 0.5
    mid = 0.5 * (a + c)
    l1 = mid + jnp.sqrt(jnp.maximum(mid * mid - det, 0.1))
    radii = jnp.where(valid, jnp.ceil(3.0 * jnp.sqrt(l1)), 0.0).astype(jnp.int32)
    opeff = jnp.where(valid, ot_ref[0:1, :], 0.0)

    rho = jnp.sqrt((px - _CX) * (px - _CX) + (py - _CY) * (py - _CY))
    rc2 = jnp.where(opeff > 0.0,
                    2.0 * l1 * jnp.log(255.0 * jnp.maximum(opeff, 1e-12)),
                    -1.0)
    rels = []
    for b in range(PB):
        dmin = jnp.maximum(jnp.maximum(_BLK_R0[b] - rho, rho - _BLK_R1[b]),
                           0.0)
        rels.append((dmin * dmin <= rc2).astype(f32))

    zero = jnp.zeros_like(px)
    params_ref[...] = jnp.concatenate(
        [px, py, conA, conB + conB, conC, opeff, tz, zero] + rels,
        axis=0).astype(f32)
    colinv_ref[...] = jnp.concatenate(
        [ct_ref[0:1, :], ct_ref[1:2, :], ct_ref[2:3, :], itz], axis=0).astype(f32)
    radii_ref[...] = radii


def _raster_kernel(counts_ref, params_ref, colinv_ref, pg_ref, bg_ref, out_ref,
                   lT_ref, te_ref):
    f32 = jnp.float32
    jc = pl.program_id(1)
    pb = pl.program_id(0)

    @pl.when(jc == 0)
    def _init():
        lT_ref[...] = jnp.zeros_like(lT_ref)
        te_ref[...] = jnp.ones_like(te_ref)
        out_ref[...] = jnp.zeros_like(out_ref)

    in_range = jc * K_BLK < counts_ref[pb]
    alive = (jnp.max(lT_ref[...]) >= LOG_EPS) & in_range

    @pl.when(alive)
    def _compute():
        px = params_ref[0:1, :]
        py = params_ref[1:2, :]
        cA = params_ref[2:3, :]
        cB2 = params_ref[3:4, :]
        cC = params_ref[4:5, :]
        op = params_ref[5:6, :]
        pgx = pg_ref[:, 0:1]
        pgy = pg_ref[:, 1:2]
        dx = px - pgx
        dy = py - pgy
        power = -0.5 * ((cA * dx + cB2 * dy) * dx + (cC * dy) * dy)
        alpha = jnp.minimum(0.99, op * jnp.exp(jnp.minimum(power, 0.0)))
        alpha = jnp.where((power > 0.0) | (alpha < 1.0 / 255.0), 0.0, alpha)
        l1m = jnp.log(1.0 - alpha)
        rowi = jax.lax.broadcasted_iota(jnp.int32, (K_BLK, K_BLK), 0)
        coli = jax.lax.broadcasted_iota(jnp.int32, (K_BLK, K_BLK), 1)
        tri = (rowi <= coli).astype(f32)
        incl = jnp.dot(l1m, tri, preferred_element_type=f32)
        lT = lT_ref[...]
        te = te_ref[...]
        cb = _bf(colinv_ref[...])
        live = lT >= LOG_EPS
        lT_end = lT + incl[:, K_BLK - 1:K_BLK]
        no_cross = jnp.min(jnp.where(live, lT_end, 0.0)) >= LOG_EPS

        @pl.when(no_cross)
        def _fast():
            tel = jnp.where(live, te, 0.0)
            wgt = alpha * tel * jnp.exp(incl - l1m)
            out_ref[...] += jnp.dot(_bf(wgt), cb, preferred_element_type=f32)
            lT_ref[...] = lT_end
            te_ref[...] = jnp.where(
                live, te * jnp.exp(incl[:, K_BLK - 1:K_BLK]), te)

        @pl.when(jnp.logical_not(no_cross))
        def _slow():
            keep = ((lT + incl) >= LOG_EPS).astype(f32)
            mval = jnp.min(jnp.where(keep > 0.0, incl, 0.0), axis=1,
                           keepdims=True)
            incl_eff = jnp.maximum(incl, mval)
            excl_eff = incl_eff - l1m * keep
            wgt = alpha * keep * te * jnp.exp(excl_eff)
            out_ref[...] += jnp.dot(_bf(wgt), cb, preferred_element_type=f32)
            lT_ref[...] = lT_end
            te_ref[...] = te * jnp.exp(incl_eff[:, K_BLK - 1:K_BLK])

    @pl.when(jc == NC - 1)
    def _finish():
        out_ref[...] += te_ref[...] * bg_ref[...]


def _project(mt, st, qt, ot, ct, vm, pm):
    return pl.pallas_call(
        _project_kernel,
        out_shape=[
            jax.ShapeDtypeStruct((8 + PB, N), jnp.float32),
            jax.ShapeDtypeStruct((4, N), jnp.float32),
            jax.ShapeDtypeStruct((1, N), jnp.int32),
        ],
        in_specs=[
            pl.BlockSpec(memory_space=pltpu.VMEM),
            pl.BlockSpec(memory_space=pltpu.VMEM),
            pl.BlockSpec(memory_space=pltpu.VMEM),
            pl.BlockSpec(memory_space=pltpu.VMEM),
            pl.BlockSpec(memory_space=pltpu.VMEM),
            pl.BlockSpec(memory_space=pltpu.SMEM),
            pl.BlockSpec(memory_space=pltpu.SMEM),
        ],
    )(mt, st, qt, ot, ct, vm, pm)


def _rasterize(counts, params_c, colinv_c, pg, bg4):
    return pl.pallas_call(
        _raster_kernel,
        grid=(PB, NC),
        in_specs=[
            pl.BlockSpec(memory_space=pltpu.SMEM),
            pl.BlockSpec((8, K_BLK), lambda pb, jc: (0, pb * NC + jc)),
            pl.BlockSpec((K_BLK, 4), lambda pb, jc: (pb * NC + jc, 0)),
            pl.BlockSpec((P_BLK, 2), lambda pb, jc: (pb, 0)),
            pl.BlockSpec((1, 4), lambda pb, jc: (0, 0)),
        ],
        out_specs=pl.BlockSpec((P_BLK, 4), lambda pb, jc: (pb, 0)),
        out_shape=jax.ShapeDtypeStruct((HW, 4), jnp.float32),
        scratch_shapes=[
            pltpu.VMEM((P_BLK, 1), jnp.float32),
            pltpu.VMEM((P_BLK, 1), jnp.float32),
        ],
        compiler_params=pltpu.CompilerParams(
            dimension_semantics=("arbitrary", "arbitrary")),
    )(counts, params_c, colinv_c, pg, bg4)


def kernel(means3D, means2D, opacities, colors_precomp, scales, rotations,
           viewmatrix, projmatrix, campos, bg):
    mt = means3D.T
    st = scales.T
    qt = rotations.T
    ot = opacities.T
    ct = colors_precomp.T
    vmb = viewmatrix.astype(jnp.bfloat16).astype(jnp.float32)
    pmb = projmatrix.astype(jnp.bfloat16).astype(jnp.float32)
    params, colinv_t, radii2 = _project(mt, st, qt, ot, ct, vmb, pmb)
    radii = radii2[0]
    order = jnp.argsort(params[6, :])
    params_s = params[:, order]
    colinv_s = colinv_t[:, order].T
    m = params_s[8:8 + PB, :] > 0.0
    pos = jnp.cumsum(m.astype(jnp.int32), axis=1) - 1
    counts = pos[:, -1] + 1
    offs = (jnp.arange(PB, dtype=jnp.int32) * N)[:, None]
    dest = jnp.where(m, pos + offs, PB * N)
    src = jnp.broadcast_to(jnp.arange(N, dtype=jnp.int32), (PB, N))
    flat_idx = jnp.full((PB * N,), N, jnp.int32).at[
        dest.reshape(-1)].set(src.reshape(-1), mode="drop")
    params_pad = jnp.pad(params_s[:8, :], ((0, 0), (0, 1)))
    colinv_pad = jnp.pad(colinv_s, ((0, 1), (0, 0)))
    params_c = params_pad[:, flat_idx]
    colinv_c = colinv_pad[flat_idx, :]
    bg4 = jnp.concatenate([bg, jnp.zeros((1,), bg.dtype)])[None, :]
    pg = jnp.asarray(_PG)
    acc = _rasterize(counts, params_c, colinv_c, pg, bg4)
    img = acc[jnp.asarray(_INV_PERM), :].reshape(H, W, 4)
    color = img[:, :, :3].transpose(2, 0, 1)
    invdepth = img[:, :, 3].reshape(1, H, W)
    return (color, radii, invdepth)


# speedup vs baseline: 1.7897x; 1.7897x over previous
import functools
import math

import jax
import jax.numpy as jnp
import numpy as np
from jax.experimental import pallas as pl
from jax.experimental.pallas import tpu as pltpu

N = 4096
H = 64
W = 64
HW = H * W
TANX = 0.5
TANY = 0.5
FX = W / (2.0 * TANX)
FY = H / (2.0 * TANY)
LIMX = 1.3 * TANX
LIMY = 1.3 * TANY
LOG_EPS = math.log(1e-4)

P_BLK = 2048
K_BLK = 256
NC = N // K_BLK
PB = HW // P_BLK

_yy, _xx = np.mgrid[0:H, 0:W]
_r2 = (_xx - (W - 1) / 2.0) ** 2 + (_yy - (H - 1) / 2.0) ** 2
_PERM = np.argsort(_r2.reshape(-1), kind="stable").astype(np.int32)
_INV_PERM = np.argsort(_PERM, kind="stable").astype(np.int32)
_PG = np.stack([(_PERM % W).astype(np.float32),
                (_PERM // W).astype(np.float32)], axis=1)
_R2S = np.sort(_r2.reshape(-1), kind="stable")
_BLK_R0 = [float(np.sqrt(_R2S[b * P_BLK])) for b in range(PB)]
_BLK_R1 = [float(np.sqrt(_R2S[(b + 1) * P_BLK - 1])) for b in range(PB)]
_CX = (W - 1) / 2.0
_CY = (H - 1) / 2.0


def _bf(x):
    return x.astype(jnp.bfloat16).astype(jnp.float32)


def _project_kernel(mt_ref, st_ref, qt_ref, ot_ref, ct_ref, vm_ref, pm_ref,
                    params_ref, colinv_ref, radii_ref):
    f32 = jnp.float32
    mx = mt_ref[0:1, :]
    my = mt_ref[1:2, :]
    mz = mt_ref[2:3, :]
    s = [st_ref[i:i + 1, :] for i in range(3)]
    qr = qt_ref[0:1, :]
    qx = qt_ref[1:2, :]
    qy = qt_ref[2:3, :]
    qz = qt_ref[3:4, :]
    nrm = jnp.sqrt(qr * qr + qx * qx + qy * qy + qz * qz) + 1e-12
    r = qr / nrm
    x = qx / nrm
    y = qy / nrm
    z = qz / nrm
    R = [
        [1 - 2 * (y * y + z * z), 2 * (x * y - r * z), 2 * (x * z + r * y)],
        [2 * (x * y + r * z), 1 - 2 * (x * x + z * z), 2 * (y * z - r * x)],
        [2 * (x * z - r * y), 2 * (y * z + r * x), 1 - 2 * (x * x + y * y)],
    ]
    M = [[_bf(R[a][j] * s[j]) for j in range(3)] for a in range(3)]
    cov3 = [[sum(M[a][j] * M[b][j] for j in range(3)) for b in range(3)]
            for a in range(3)]

    vm = [[vm_ref[i, j] for j in range(4)] for i in range(4)]
    pm = [[pm_ref[i, j] for j in range(4)] for i in range(4)]
    mxb, myb, mzb = _bf(mx), _bf(my), _bf(mz)
    tx = vm[0][0] * mxb + vm[0][1] * myb + vm[0][2] * mzb + vm[0][3]
    ty = vm[1][0] * mxb + vm[1][1] * myb + vm[1][2] * mzb + vm[1][3]
    tz = vm[2][0] * mxb + vm[2][1] * myb + vm[2][2] * mzb + vm[2][3]
    ph0 = pm[0][0] * mxb + pm[0][1] * myb + pm[0][2] * mzb + pm[0][3]
    ph1 = pm[1][0] * mxb + pm[1][1] * myb + pm[1][2] * mzb + pm[1][3]
    ph3 = pm[3][0] * mxb + pm[3][1] * myb + pm[3][2] * mzb + pm[3][3]
    pw = 1.0 / (ph3 + 1e-7)
    ppx = ph0 * pw
    ppy = ph1 * pw

    tzc = jnp.where(jnp.abs(tz) < 1e-6, 1e-6, tz)
    txc = jnp.clip(tx / tzc, -LIMX, LIMX) * tzc
    tyc = jnp.clip(ty / tzc, -LIMY, LIMY) * tzc
    itz = 1.0 / tzc
    J0 = [_bf(FX / tzc), jnp.zeros_like(itz), _bf(-FX * txc / (tzc * tzc))]
    J1 = [jnp.zeros_like(itz), _bf(FY / tzc), _bf(-FY * tyc / (tzc * tzc))]
    Tm0 = [J0[0] * vm[0][k] + J0[2] * vm[2][k] for k in range(3)]
    Tm1 = [J1[1] * vm[1][k] + J1[2] * vm[2][k] for k in range(3)]
    Tm0b = [_bf(t) for t in Tm0]
    Tm1b = [_bf(t) for t in Tm1]
    cov3b = [[_bf(cov3[a][b]) for b in range(3)] for a in range(3)]
    u0 = [_bf(sum(Tm0b[k] * cov3b[k][j] for k in range(3))) for j in range(3)]
    u1 = [_bf(sum(Tm1b[k] * cov3b[k][j] for k in range(3))) for j in range(3)]
    cov00 = sum(u0[j] * Tm0b[j] for j in range(3))
    cov01 = sum(u0[j] * Tm1b[j] for j in range(3))
    cov11 = sum(u1[j] * Tm1b[j] for j in range(3))

    a = cov00 + 0.3
    b = cov01
    c = cov11 + 0.3
    det = a * c - b * b
    valid = (det > 0.0) & (tz > 0.2)
    det_safe = jnp.where(valid, det, 1.0)
    conA = c / det_safe
    conB = -b / det_safe
    conC = a / det_safe
    px = ((ppx + 1.0) * W - 1.0) * 0.5
    py = ((ppy + 1.0) * H - 1.0) * 0.5
    mid = 0.5 * (a + c)
    l1 = mid + jnp.sqrt(jnp.maximum(mid * mid - det, 0.1))
    radii = jnp.where(valid, jnp.ceil(3.0 * jnp.sqrt(l1)), 0.0).astype(jnp.int32)
    opeff = jnp.where(valid, ot_ref[0:1, :], 0.0)

    zero = jnp.zeros_like(px)
    params_ref[...] = jnp.concatenate(
        [px, py, conA, conB + conB, conC, opeff, tz, zero],
        axis=0).astype(f32)
    colinv_ref[...] = jnp.concatenate(
        [ct_ref[0:1, :], ct_ref[1:2, :], ct_ref[2:3, :], itz], axis=0).astype(f32)
    radii_ref[...] = radii


def _raster_kernel(params_ref, colinv_ref, pg_ref, bg_ref, out_ref,
                   lT_ref, te_ref):
    f32 = jnp.float32
    jc = pl.program_id(1)
    pb = pl.program_id(0)

    @pl.when(jc == 0)
    def _init():
        lT_ref[...] = jnp.zeros_like(lT_ref)
        te_ref[...] = jnp.ones_like(te_ref)
        out_ref[...] = jnp.zeros_like(out_ref)

    alive = jnp.max(lT_ref[...]) >= LOG_EPS

    @pl.when(alive)
    def _compute():
        px = params_ref[0:1, :]
        py = params_ref[1:2, :]
        cA = params_ref[2:3, :]
        cB2 = params_ref[3:4, :]
        cC = params_ref[4:5, :]
        op = params_ref[5:6, :]
        pgx = pg_ref[:, 0:1]
        pgy = pg_ref[:, 1:2]
        dx = px - pgx
        dy = py - pgy
        power = -0.5 * ((cA * dx + cB2 * dy) * dx + (cC * dy) * dy)
        alpha = jnp.minimum(0.99, op * jnp.exp(jnp.minimum(power, 0.0)))
        alpha = jnp.where((power > 0.0) | (alpha < 1.0 / 255.0), 0.0, alpha)
        l1m = jnp.log(1.0 - alpha)
        rowi = jax.lax.broadcasted_iota(jnp.int32, (K_BLK, K_BLK), 0)
        coli = jax.lax.broadcasted_iota(jnp.int32, (K_BLK, K_BLK), 1)
        tri = (rowi <= coli).astype(f32)
        incl = jnp.dot(l1m, tri, preferred_element_type=f32)
        lT = lT_ref[...]
        te = te_ref[...]
        cb = _bf(colinv_ref[...])
        live = lT >= LOG_EPS
        lT_end = lT + incl[:, K_BLK - 1:K_BLK]
        no_cross = jnp.min(jnp.where(live, lT_end, 0.0)) >= LOG_EPS

        @pl.when(no_cross)
        def _fast():
            tel = jnp.where(live, te, 0.0)
            wgt = alpha * tel * jnp.exp(incl - l1m)
            out_ref[...] += jnp.dot(_bf(wgt), cb, preferred_element_type=f32)
            lT_ref[...] = lT_end
            te_ref[...] = jnp.where(
                live, te * jnp.exp(incl[:, K_BLK - 1:K_BLK]), te)

        @pl.when(jnp.logical_not(no_cross))
        def _slow():
            keep = ((lT + incl) >= LOG_EPS).astype(f32)
            mval = jnp.min(jnp.where(keep > 0.0, incl, 0.0), axis=1,
                           keepdims=True)
            incl_eff = jnp.maximum(incl, mval)
            excl_eff = incl_eff - l1m * keep
            wgt = alpha * keep * te * jnp.exp(excl_eff)
            out_ref[...] += jnp.dot(_bf(wgt), cb, preferred_element_type=f32)
            lT_ref[...] = lT_end
            te_ref[...] = te * jnp.exp(incl_eff[:, K_BLK - 1:K_BLK])

    @pl.when(jc == NC - 1)
    def _finish():
        out_ref[...] += te_ref[...] * bg_ref[...]


def _project(mt, st, qt, ot, ct, vm, pm):
    return pl.pallas_call(
        _project_kernel,
        out_shape=[
            jax.ShapeDtypeStruct((8, N), jnp.float32),
            jax.ShapeDtypeStruct((4, N), jnp.float32),
            jax.ShapeDtypeStruct((1, N), jnp.int32),
        ],
        in_specs=[
            pl.BlockSpec(memory_space=pltpu.VMEM),
            pl.BlockSpec(memory_space=pltpu.VMEM),
            pl.BlockSpec(memory_space=pltpu.VMEM),
            pl.BlockSpec(memory_space=pltpu.VMEM),
            pl.BlockSpec(memory_space=pltpu.VMEM),
            pl.BlockSpec(memory_space=pltpu.SMEM),
            pl.BlockSpec(memory_space=pltpu.SMEM),
        ],
    )(mt, st, qt, ot, ct, vm, pm)


def _rasterize(params_s, colinv_s, pg, bg4):
    return pl.pallas_call(
        _raster_kernel,
        grid=(PB, NC),
        in_specs=[
            pl.BlockSpec((8, K_BLK), lambda pb, jc: (0, jc)),
            pl.BlockSpec((K_BLK, 4), lambda pb, jc: (jc, 0)),
            pl.BlockSpec((P_BLK, 2), lambda pb, jc: (pb, 0)),
            pl.BlockSpec((1, 4), lambda pb, jc: (0, 0)),
        ],
        out_specs=pl.BlockSpec((P_BLK, 4), lambda pb, jc: (pb, 0)),
        out_shape=jax.ShapeDtypeStruct((HW, 4), jnp.float32),
        scratch_shapes=[
            pltpu.VMEM((P_BLK, 1), jnp.float32),
            pltpu.VMEM((P_BLK, 1), jnp.float32),
        ],
        compiler_params=pltpu.CompilerParams(
            dimension_semantics=("arbitrary", "arbitrary")),
    )(params_s, colinv_s, pg, bg4)


def kernel(means3D, means2D, opacities, colors_precomp, scales, rotations,
           viewmatrix, projmatrix, campos, bg):
    mt = means3D.T
    st = scales.T
    qt = rotations.T
    ot = opacities.T
    ct = colors_precomp.T
    vmb = viewmatrix.astype(jnp.bfloat16).astype(jnp.float32)
    pmb = projmatrix.astype(jnp.bfloat16).astype(jnp.float32)
    params, colinv_t, radii2 = _project(mt, st, qt, ot, ct, vmb, pmb)
    radii = radii2[0]
    order = jnp.argsort(params[6, :])
    params_s = params[:, order]
    colinv_s = colinv_t[:, order].T
    bg4 = jnp.concatenate([bg, jnp.zeros((1,), bg.dtype)])[None, :]
    pg = jnp.asarray(_PG)
    acc = _rasterize(params_s, colinv_s, pg, bg4)
    img = acc[jnp.asarray(_INV_PERM), :].reshape(H, W, 4)
    color = img[:, :, :3].transpose(2, 0, 1)
    invdepth = img[:, :, 3].reshape(1, H, W)
    return (color, radii, invdepth)


# speedup vs baseline: 1.8372x; 1.0265x over previous
import functools
import math

import jax
import jax.numpy as jnp
import numpy as np
from jax.experimental import pallas as pl
from jax.experimental.pallas import tpu as pltpu

N = 4096
H = 64
W = 64
HW = H * W
TANX = 0.5
TANY = 0.5
FX = W / (2.0 * TANX)
FY = H / (2.0 * TANY)
LIMX = 1.3 * TANX
LIMY = 1.3 * TANY
LOG_EPS = math.log(1e-4)

P_BLK = 4096
K_BLK = 256
NC = N // K_BLK
PB = HW // P_BLK

_yy, _xx = np.mgrid[0:H, 0:W]
_r2 = (_xx - (W - 1) / 2.0) ** 2 + (_yy - (H - 1) / 2.0) ** 2
_PERM = np.argsort(_r2.reshape(-1), kind="stable").astype(np.int32)
_INV_PERM = np.argsort(_PERM, kind="stable").astype(np.int32)
_PG = np.stack([(_PERM % W).astype(np.float32),
                (_PERM // W).astype(np.float32)], axis=1)
_R2S = np.sort(_r2.reshape(-1), kind="stable")
_BLK_R0 = [float(np.sqrt(_R2S[b * P_BLK])) for b in range(PB)]
_BLK_R1 = [float(np.sqrt(_R2S[(b + 1) * P_BLK - 1])) for b in range(PB)]
_CX = (W - 1) / 2.0
_CY = (H - 1) / 2.0


def _bf(x):
    return x.astype(jnp.bfloat16).astype(jnp.float32)


def _project_kernel(mt_ref, st_ref, qt_ref, ot_ref, ct_ref, vm_ref, pm_ref,
                    params_ref, colinv_ref, radii_ref):
    f32 = jnp.float32
    mx = mt_ref[0:1, :]
    my = mt_ref[1:2, :]
    mz = mt_ref[2:3, :]
    s = [st_ref[i:i + 1, :] for i in range(3)]
    qr = qt_ref[0:1, :]
    qx = qt_ref[1:2, :]
    qy = qt_ref[2:3, :]
    qz = qt_ref[3:4, :]
    nrm = jnp.sqrt(qr * qr + qx * qx + qy * qy + qz * qz) + 1e-12
    r = qr / nrm
    x = qx / nrm
    y = qy / nrm
    z = qz / nrm
    R = [
        [1 - 2 * (y * y + z * z), 2 * (x * y - r * z), 2 * (x * z + r * y)],
        [2 * (x * y + r * z), 1 - 2 * (x * x + z * z), 2 * (y * z - r * x)],
        [2 * (x * z - r * y), 2 * (y * z + r * x), 1 - 2 * (x * x + y * y)],
    ]
    M = [[_bf(R[a][j] * s[j]) for j in range(3)] for a in range(3)]
    cov3 = [[sum(M[a][j] * M[b][j] for j in range(3)) for b in range(3)]
            for a in range(3)]

    vm = [[vm_ref[i, j] for j in range(4)] for i in range(4)]
    pm = [[pm_ref[i, j] for j in range(4)] for i in range(4)]
    mxb, myb, mzb = _bf(mx), _bf(my), _bf(mz)
    tx = vm[0][0] * mxb + vm[0][1] * myb + vm[0][2] * mzb + vm[0][3]
    ty = vm[1][0] * mxb + vm[1][1] * myb + vm[1][2] * mzb + vm[1][3]
    tz = vm[2][0] * mxb + vm[2][1] * myb + vm[2][2] * mzb + vm[2][3]
    ph0 = pm[0][0] * mxb + pm[0][1] * myb + pm[0][2] * mzb + pm[0][3]
    ph1 = pm[1][0] * mxb + pm[1][1] * myb + pm[1][2] * mzb + pm[1][3]
    ph3 = pm[3][0] * mxb + pm[3][1] * myb + pm[3][2] * mzb + pm[3][3]
    pw = 1.0 / (ph3 + 1e-7)
    ppx = ph0 * pw
    ppy = ph1 * pw

    tzc = jnp.where(jnp.abs(tz) < 1e-6, 1e-6, tz)
    txc = jnp.clip(tx / tzc, -LIMX, LIMX) * tzc
    tyc = jnp.clip(ty / tzc, -LIMY, LIMY) * tzc
    itz = 1.0 / tzc
    J0 = [_bf(FX / tzc), jnp.zeros_like(itz), _bf(-FX * txc / (tzc * tzc))]
    J1 = [jnp.zeros_like(itz), _bf(FY / tzc), _bf(-FY * tyc / (tzc * tzc))]
    Tm0 = [J0[0] * vm[0][k] + J0[2] * vm[2][k] for k in range(3)]
    Tm1 = [J1[1] * vm[1][k] + J1[2] * vm[2][k] for k in range(3)]
    Tm0b = [_bf(t) for t in Tm0]
    Tm1b = [_bf(t) for t in Tm1]
    cov3b = [[_bf(cov3[a][b]) for b in range(3)] for a in range(3)]
    u0 = [_bf(sum(Tm0b[k] * cov3b[k][j] for k in range(3))) for j in range(3)]
    u1 = [_bf(sum(Tm1b[k] * cov3b[k][j] for k in range(3))) for j in range(3)]
    cov00 = sum(u0[j] * Tm0b[j] for j in range(3))
    cov01 = sum(u0[j] * Tm1b[j] for j in range(3))
    cov11 = sum(u1[j] * Tm1b[j] for j in range(3))

    a = cov00 + 0.3
    b = cov01
    c = cov11 + 0.3
    det = a * c - b * b
    valid = (det > 0.0) & (tz > 0.2)
    det_safe = jnp.where(valid, det, 1.0)
    conA = c / det_safe
    conB = -b / det_safe
    conC = a / det_safe
    px = ((ppx + 1.0) * W - 1.0) * 0.5
    py = ((ppy + 1.0) * H - 1.0) * 0.5
    mid = 0.5 * (a + c)
    l1 = mid + jnp.sqrt(jnp.maximum(mid * mid - det, 0.1))
    radii = jnp.where(valid, jnp.ceil(3.0 * jnp.sqrt(l1)), 0.0).astype(jnp.int32)
    opeff = jnp.where(valid, ot_ref[0:1, :], 0.0)

    zero = jnp.zeros_like(px)
    params_ref[...] = jnp.concatenate(
        [px, py, conA, conB + conB, conC, opeff, tz, zero],
        axis=0).astype(f32)
    colinv_ref[...] = jnp.concatenate(
        [ct_ref[0:1, :], ct_ref[1:2, :], ct_ref[2:3, :], itz], axis=0).astype(f32)
    radii_ref[...] = radii


def _raster_kernel(params_ref, colinv_ref, pg_ref, bg_ref, out_ref,
                   lT_ref, te_ref):
    f32 = jnp.float32
    jc = pl.program_id(1)
    pb = pl.program_id(0)

    @pl.when(jc == 0)
    def _init():
        lT_ref[...] = jnp.zeros_like(lT_ref)
        te_ref[...] = jnp.ones_like(te_ref)
        out_ref[...] = jnp.zeros_like(out_ref)

    alive = jnp.max(lT_ref[...]) >= LOG_EPS

    @pl.when(alive)
    def _compute():
        px = params_ref[0:1, :]
        py = params_ref[1:2, :]
        cA = params_ref[2:3, :]
        cB2 = params_ref[3:4, :]
        cC = params_ref[4:5, :]
        op = params_ref[5:6, :]
        pgx = pg_ref[:, 0:1]
        pgy = pg_ref[:, 1:2]
        dx = px - pgx
        dy = py - pgy
        power = -0.5 * ((cA * dx + cB2 * dy) * dx + (cC * dy) * dy)
        alpha = jnp.minimum(0.99, op * jnp.exp(jnp.minimum(power, 0.0)))
        alpha = jnp.where((power > 0.0) | (alpha < 1.0 / 255.0), 0.0, alpha)
        l1m = jnp.log(1.0 - alpha)
        rowi = jax.lax.broadcasted_iota(jnp.int32, (K_BLK, K_BLK), 0)
        coli = jax.lax.broadcasted_iota(jnp.int32, (K_BLK, K_BLK), 1)
        tri = (rowi <= coli).astype(f32)
        incl = jnp.dot(l1m, tri, preferred_element_type=f32)
        lT = lT_ref[...]
        te = te_ref[...]
        cb = _bf(colinv_ref[...])
        live = lT >= LOG_EPS
        lT_end = lT + incl[:, K_BLK - 1:K_BLK]
        no_cross = jnp.min(jnp.where(live, lT_end, 0.0)) >= LOG_EPS

        @pl.when(no_cross)
        def _fast():
            tel = jnp.where(live, te, 0.0)
            wgt = alpha * tel * jnp.exp(incl - l1m)
            out_ref[...] += jnp.dot(_bf(wgt), cb, preferred_element_type=f32)
            lT_ref[...] = lT_end
            te_ref[...] = jnp.where(
                live, te * jnp.exp(incl[:, K_BLK - 1:K_BLK]), te)

        @pl.when(jnp.logical_not(no_cross))
        def _slow():
            keep = ((lT + incl) >= LOG_EPS).astype(f32)
            mval = jnp.min(jnp.where(keep > 0.0, incl, 0.0), axis=1,
                           keepdims=True)
            incl_eff = jnp.maximum(incl, mval)
            excl_eff = incl_eff - l1m * keep
            wgt = alpha * keep * te * jnp.exp(excl_eff)
            out_ref[...] += jnp.dot(_bf(wgt), cb, preferred_element_type=f32)
            lT_ref[...] = lT_end
            te_ref[...] = te * jnp.exp(incl_eff[:, K_BLK - 1:K_BLK])

    @pl.when(jc == NC - 1)
    def _finish():
        out_ref[...] += te_ref[...] * bg_ref[...]


def _project(mt, st, qt, ot, ct, vm, pm):
    return pl.pallas_call(
        _project_kernel,
        out_shape=[
            jax.ShapeDtypeStruct((8, N), jnp.float32),
            jax.ShapeDtypeStruct((4, N), jnp.float32),
            jax.ShapeDtypeStruct((1, N), jnp.int32),
        ],
        in_specs=[
            pl.BlockSpec(memory_space=pltpu.VMEM),
            pl.BlockSpec(memory_space=pltpu.VMEM),
            pl.BlockSpec(memory_space=pltpu.VMEM),
            pl.BlockSpec(memory_space=pltpu.VMEM),
            pl.BlockSpec(memory_space=pltpu.VMEM),
            pl.BlockSpec(memory_space=pltpu.SMEM),
            pl.BlockSpec(memory_space=pltpu.SMEM),
        ],
    )(mt, st, qt, ot, ct, vm, pm)


def _rasterize(params_s, colinv_s, pg, bg4):
    return pl.pallas_call(
        _raster_kernel,
        grid=(PB, NC),
        in_specs=[
            pl.BlockSpec((8, K_BLK), lambda pb, jc: (0, jc)),
            pl.BlockSpec((K_BLK, 4), lambda pb, jc: (jc, 0)),
            pl.BlockSpec((P_BLK, 2), lambda pb, jc: (pb, 0)),
            pl.BlockSpec((1, 4), lambda pb, jc: (0, 0)),
        ],
        out_specs=pl.BlockSpec((P_BLK, 4), lambda pb, jc: (pb, 0)),
        out_shape=jax.ShapeDtypeStruct((HW, 4), jnp.float32),
        scratch_shapes=[
            pltpu.VMEM((P_BLK, 1), jnp.float32),
            pltpu.VMEM((P_BLK, 1), jnp.float32),
        ],
        compiler_params=pltpu.CompilerParams(
            dimension_semantics=("arbitrary", "arbitrary")),
    )(params_s, colinv_s, pg, bg4)


def kernel(means3D, means2D, opacities, colors_precomp, scales, rotations,
           viewmatrix, projmatrix, campos, bg):
    mt = means3D.T
    st = scales.T
    qt = rotations.T
    ot = opacities.T
    ct = colors_precomp.T
    vmb = viewmatrix.astype(jnp.bfloat16).astype(jnp.float32)
    pmb = projmatrix.astype(jnp.bfloat16).astype(jnp.float32)
    params, colinv_t, radii2 = _project(mt, st, qt, ot, ct, vmb, pmb)
    radii = radii2[0]
    order = jnp.argsort(params[6, :])
    params_s = params[:, order]
    colinv_s = colinv_t[:, order].T
    bg4 = jnp.concatenate([bg, jnp.zeros((1,), bg.dtype)])[None, :]
    pg = jnp.asarray(_PG)
    acc = _rasterize(params_s, colinv_s, pg, bg4)
    img = acc[jnp.asarray(_INV_PERM), :].reshape(H, W, 4)
    color = img[:, :, :3].transpose(2, 0, 1)
    invdepth = img[:, :, 3].reshape(1, H, W)
    return (color, radii, invdepth)


# speedup vs baseline: 1.8832x; 1.0250x over previous
import functools
import math

import jax
import jax.numpy as jnp
import numpy as np
from jax.experimental import pallas as pl
from jax.experimental.pallas import tpu as pltpu

N = 4096
H = 64
W = 64
HW = H * W
TANX = 0.5
TANY = 0.5
FX = W / (2.0 * TANX)
FY = H / (2.0 * TANY)
LIMX = 1.3 * TANX
LIMY = 1.3 * TANY
LOG_EPS = math.log(1e-4)

P_BLK = 4096
K_BLK = 256
NC = N // K_BLK
PB = HW // P_BLK

_yy, _xx = np.mgrid[0:H, 0:W]
_PG = np.stack([_xx.reshape(-1).astype(np.float32),
                _yy.reshape(-1).astype(np.float32)], axis=1)


def _bf(x):
    return x.astype(jnp.bfloat16).astype(jnp.float32)


def _project_kernel(mt_ref, st_ref, qt_ref, ot_ref, ct_ref, vm_ref, pm_ref,
                    params_ref, colinv_ref, radii_ref):
    f32 = jnp.float32
    mx = mt_ref[0:1, :]
    my = mt_ref[1:2, :]
    mz = mt_ref[2:3, :]
    s = [st_ref[i:i + 1, :] for i in range(3)]
    qr = qt_ref[0:1, :]
    qx = qt_ref[1:2, :]
    qy = qt_ref[2:3, :]
    qz = qt_ref[3:4, :]
    nrm = jnp.sqrt(qr * qr + qx * qx + qy * qy + qz * qz) + 1e-12
    r = qr / nrm
    x = qx / nrm
    y = qy / nrm
    z = qz / nrm
    R = [
        [1 - 2 * (y * y + z * z), 2 * (x * y - r * z), 2 * (x * z + r * y)],
        [2 * (x * y + r * z), 1 - 2 * (x * x + z * z), 2 * (y * z - r * x)],
        [2 * (x * z - r * y), 2 * (y * z + r * x), 1 - 2 * (x * x + y * y)],
    ]
    M = [[_bf(R[a][j] * s[j]) for j in range(3)] for a in range(3)]
    cov3 = [[sum(M[a][j] * M[b][j] for j in range(3)) for b in range(3)]
            for a in range(3)]

    vm = [[vm_ref[i, j] for j in range(4)] for i in range(4)]
    pm = [[pm_ref[i, j] for j in range(4)] for i in range(4)]
    mxb, myb, mzb = _bf(mx), _bf(my), _bf(mz)
    tx = vm[0][0] * mxb + vm[0][1] * myb + vm[0][2] * mzb + vm[0][3]
    ty = vm[1][0] * mxb + vm[1][1] * myb + vm[1][2] * mzb + vm[1][3]
    tz = vm[2][0] * mxb + vm[2][1] * myb + vm[2][2] * mzb + vm[2][3]
    ph0 = pm[0][0] * mxb + pm[0][1] * myb + pm[0][2] * mzb + pm[0][3]
    ph1 = pm[1][0] * mxb + pm[1][1] * myb + pm[1][2] * mzb + pm[1][3]
    ph3 = pm[3][0] * mxb + pm[3][1] * myb + pm[3][2] * mzb + pm[3][3]
    pw = 1.0 / (ph3 + 1e-7)
    ppx = ph0 * pw
    ppy = ph1 * pw

    tzc = jnp.where(jnp.abs(tz) < 1e-6, 1e-6, tz)
    txc = jnp.clip(tx / tzc, -LIMX, LIMX) * tzc
    tyc = jnp.clip(ty / tzc, -LIMY, LIMY) * tzc
    itz = 1.0 / tzc
    J0 = [_bf(FX / tzc), jnp.zeros_like(itz), _bf(-FX * txc / (tzc * tzc))]
    J1 = [jnp.zeros_like(itz), _bf(FY / tzc), _bf(-FY * tyc / (tzc * tzc))]
    Tm0 = [J0[0] * vm[0][k] + J0[2] * vm[2][k] for k in range(3)]
    Tm1 = [J1[1] * vm[1][k] + J1[2] * vm[2][k] for k in range(3)]
    Tm0b = [_bf(t) for t in Tm0]
    Tm1b = [_bf(t) for t in Tm1]
    cov3b = [[_bf(cov3[a][b]) for b in range(3)] for a in range(3)]
    u0 = [_bf(sum(Tm0b[k] * cov3b[k][j] for k in range(3))) for j in range(3)]
    u1 = [_bf(sum(Tm1b[k] * cov3b[k][j] for k in range(3))) for j in range(3)]
    cov00 = sum(u0[j] * Tm0b[j] for j in range(3))
    cov01 = sum(u0[j] * Tm1b[j] for j in range(3))
    cov11 = sum(u1[j] * Tm1b[j] for j in range(3))

    a = cov00 + 0.3
    b = cov01
    c = cov11 + 0.3
    det = a * c - b * b
    valid = (det > 0.0) & (tz > 0.2)
    det_safe = jnp.where(valid, det, 1.0)
    conA = c / det_safe
    conB = -b / det_safe
    conC = a / det_safe
    px = ((ppx + 1.0) * W - 1.0) * 0.5
    py = ((ppy + 1.0) * H - 1.0) * 0.5
    mid = 0.5 * (a + c)
    l1 = mid + jnp.sqrt(jnp.maximum(mid * mid - det, 0.1))
    radii = jnp.where(valid, jnp.ceil(3.0 * jnp.sqrt(l1)), 0.0).astype(jnp.int32)
    opeff = jnp.where(valid, ot_ref[0:1, :], 0.0)

    zero = jnp.zeros_like(px)
    params_ref[...] = jnp.concatenate(
        [px, py, conA, conB + conB, conC, opeff, tz, zero],
        axis=0).astype(f32)
    colinv_ref[...] = jnp.concatenate(
        [ct_ref[0:1, :], ct_ref[1:2, :], ct_ref[2:3, :], itz], axis=0).astype(f32)
    radii_ref[...] = radii


def _raster_kernel(params_ref, colinv_ref, pg_ref, bg_ref, out_ref,
                   lT_ref, te_ref):
    f32 = jnp.float32
    jc = pl.program_id(1)
    pb = pl.program_id(0)

    @pl.when(jc == 0)
    def _init():
        lT_ref[...] = jnp.zeros_like(lT_ref)
        te_ref[...] = jnp.ones_like(te_ref)
        out_ref[...] = jnp.zeros_like(out_ref)

    alive = jnp.max(lT_ref[...]) >= LOG_EPS

    @pl.when(alive)
    def _compute():
        px = params_ref[0:1, :]
        py = params_ref[1:2, :]
        cA = params_ref[2:3, :]
        cB2 = params_ref[3:4, :]
        cC = params_ref[4:5, :]
        op = params_ref[5:6, :]
        pgx = pg_ref[:, 0:1]
        pgy = pg_ref[:, 1:2]
        dx = px - pgx
        dy = py - pgy
        power = -0.5 * ((cA * dx + cB2 * dy) * dx + (cC * dy) * dy)
        alpha = jnp.minimum(0.99, op * jnp.exp(jnp.minimum(power, 0.0)))
        alpha = jnp.where((power > 0.0) | (alpha < 1.0 / 255.0), 0.0, alpha)
        l1m = jnp.log(1.0 - alpha)
        rowi = jax.lax.broadcasted_iota(jnp.int32, (K_BLK, K_BLK), 0)
        coli = jax.lax.broadcasted_iota(jnp.int32, (K_BLK, K_BLK), 1)
        tri = (rowi <= coli).astype(f32)
        incl = jnp.dot(l1m, tri, preferred_element_type=f32)
        lT = lT_ref[...]
        te = te_ref[...]
        cb = _bf(colinv_ref[...])
        live = lT >= LOG_EPS
        lT_end = lT + incl[:, K_BLK - 1:K_BLK]
        no_cross = jnp.min(jnp.where(live, lT_end, 0.0)) >= LOG_EPS

        @pl.when(no_cross)
        def _fast():
            tel = jnp.where(live, te, 0.0)
            wgt = alpha * tel * jnp.exp(incl - l1m)
            out_ref[...] += jnp.dot(_bf(wgt), cb, preferred_element_type=f32)
            lT_ref[...] = lT_end
            te_ref[...] = jnp.where(
                live, te * jnp.exp(incl[:, K_BLK - 1:K_BLK]), te)

        @pl.when(jnp.logical_not(no_cross))
        def _slow():
            keep = ((lT + incl) >= LOG_EPS).astype(f32)
            mval = jnp.min(jnp.where(keep > 0.0, incl, 0.0), axis=1,
                           keepdims=True)
            incl_eff = jnp.maximum(incl, mval)
            excl_eff = incl_eff - l1m * keep
            wgt = alpha * keep * te * jnp.exp(excl_eff)
            out_ref[...] += jnp.dot(_bf(wgt), cb, preferred_element_type=f32)
            lT_ref[...] = lT_end
            te_ref[...] = te * jnp.exp(incl_eff[:, K_BLK - 1:K_BLK])

    @pl.when(jc == NC - 1)
    def _finish():
        out_ref[...] += te_ref[...] * bg_ref[...]


def _project(mt, st, qt, ot, ct, vm, pm):
    return pl.pallas_call(
        _project_kernel,
        out_shape=[
            jax.ShapeDtypeStruct((8, N), jnp.float32),
            jax.ShapeDtypeStruct((4, N), jnp.float32),
            jax.ShapeDtypeStruct((1, N), jnp.int32),
        ],
        in_specs=[
            pl.BlockSpec(memory_space=pltpu.VMEM),
            pl.BlockSpec(memory_space=pltpu.VMEM),
            pl.BlockSpec(memory_space=pltpu.VMEM),
            pl.BlockSpec(memory_space=pltpu.VMEM),
            pl.BlockSpec(memory_space=pltpu.VMEM),
            pl.BlockSpec(memory_space=pltpu.SMEM),
            pl.BlockSpec(memory_space=pltpu.SMEM),
        ],
    )(mt, st, qt, ot, ct, vm, pm)


def _rasterize(params_s, colinv_s, pg, bg4):
    return pl.pallas_call(
        _raster_kernel,
        grid=(PB, NC),
        in_specs=[
            pl.BlockSpec((8, K_BLK), lambda pb, jc: (0, jc)),
            pl.BlockSpec((K_BLK, 4), lambda pb, jc: (jc, 0)),
            pl.BlockSpec((P_BLK, 2), lambda pb, jc: (pb, 0)),
            pl.BlockSpec((1, 4), lambda pb, jc: (0, 0)),
        ],
        out_specs=pl.BlockSpec((P_BLK, 4), lambda pb, jc: (pb, 0)),
        out_shape=jax.ShapeDtypeStruct((HW, 4), jnp.float32),
        scratch_shapes=[
            pltpu.VMEM((P_BLK, 1), jnp.float32),
            pltpu.VMEM((P_BLK, 1), jnp.float32),
        ],
        compiler_params=pltpu.CompilerParams(
            dimension_semantics=("arbitrary", "arbitrary")),
    )(params_s, colinv_s, pg, bg4)


def kernel(means3D, means2D, opacities, colors_precomp, scales, rotations,
           viewmatrix, projmatrix, campos, bg):
    mt = means3D.T
    st = scales.T
    qt = rotations.T
    ot = opacities.T
    ct = colors_precomp.T
    vmb = viewmatrix.astype(jnp.bfloat16).astype(jnp.float32)
    pmb = projmatrix.astype(jnp.bfloat16).astype(jnp.float32)
    params, colinv_t, radii2 = _project(mt, st, qt, ot, ct, vmb, pmb)
    radii = radii2[0]
    order = jnp.argsort(params[6, :])
    params_s = params[:, order]
    colinv_s = colinv_t[:, order].T
    bg4 = jnp.concatenate([bg, jnp.zeros((1,), bg.dtype)])[None, :]
    pg = jnp.asarray(_PG)
    acc = _rasterize(params_s, colinv_s, pg, bg4)
    img = acc.reshape(H, W, 4)
    color = img[:, :, :3].transpose(2, 0, 1)
    invdepth = img[:, :, 3].reshape(1, H, W)
    return (color, radii, invdepth)


# speedup vs baseline: 2.0753x; 1.1020x over previous
import functools
import math

import jax
import jax.numpy as jnp
import numpy as np
from jax.experimental import pallas as pl
from jax.experimental.pallas import tpu as pltpu

N = 4096
H = 64
W = 64
HW = H * W
TANX = 0.5
TANY = 0.5
FX = W / (2.0 * TANX)
FY = H / (2.0 * TANY)
LIMX = 1.3 * TANX
LIMY = 1.3 * TANY
LOG_EPS = math.log(1e-4)

P_BLK = 4096
K_BLK = 512
NC = N // K_BLK
PB = HW // P_BLK

_yy, _xx = np.mgrid[0:H, 0:W]
_PG = np.stack([_xx.reshape(-1).astype(np.float32),
                _yy.reshape(-1).astype(np.float32)], axis=1)


def _bf(x):
    return x.astype(jnp.bfloat16).astype(jnp.float32)


def _project_kernel(mt_ref, st_ref, qt_ref, ot_ref, ct_ref, vm_ref, pm_ref,
                    params_ref, colinv_ref, radii_ref):
    f32 = jnp.float32
    mx = mt_ref[0:1, :]
    my = mt_ref[1:2, :]
    mz = mt_ref[2:3, :]
    s = [st_ref[i:i + 1, :] for i in range(3)]
    qr = qt_ref[0:1, :]
    qx = qt_ref[1:2, :]
    qy = qt_ref[2:3, :]
    qz = qt_ref[3:4, :]
    nrm = jnp.sqrt(qr * qr + qx * qx + qy * qy + qz * qz) + 1e-12
    r = qr / nrm
    x = qx / nrm
    y = qy / nrm
    z = qz / nrm
    R = [
        [1 - 2 * (y * y + z * z), 2 * (x * y - r * z), 2 * (x * z + r * y)],
        [2 * (x * y + r * z), 1 - 2 * (x * x + z * z), 2 * (y * z - r * x)],
        [2 * (x * z - r * y), 2 * (y * z + r * x), 1 - 2 * (x * x + y * y)],
    ]
    M = [[_bf(R[a][j] * s[j]) for j in range(3)] for a in range(3)]
    cov3 = [[sum(M[a][j] * M[b][j] for j in range(3)) for b in range(3)]
            for a in range(3)]

    vm = [[vm_ref[i, j] for j in range(4)] for i in range(4)]
    pm = [[pm_ref[i, j] for j in range(4)] for i in range(4)]
    mxb, myb, mzb = _bf(mx), _bf(my), _bf(mz)
    tx = vm[0][0] * mxb + vm[0][1] * myb + vm[0][2] * mzb + vm[0][3]
    ty = vm[1][0] * mxb + vm[1][1] * myb + vm[1][2] * mzb + vm[1][3]
    tz = vm[2][0] * mxb + vm[2][1] * myb + vm[2][2] * mzb + vm[2][3]
    ph0 = pm[0][0] * mxb + pm[0][1] * myb + pm[0][2] * mzb + pm[0][3]
    ph1 = pm[1][0] * mxb + pm[1][1] * myb + pm[1][2] * mzb + pm[1][3]
    ph3 = pm[3][0] * mxb + pm[3][1] * myb + pm[3][2] * mzb + pm[3][3]
    pw = 1.0 / (ph3 + 1e-7)
    ppx = ph0 * pw
    ppy = ph1 * pw

    tzc = jnp.where(jnp.abs(tz) < 1e-6, 1e-6, tz)
    txc = jnp.clip(tx / tzc, -LIMX, LIMX) * tzc
    tyc = jnp.clip(ty / tzc, -LIMY, LIMY) * tzc
    itz = 1.0 / tzc
    J0 = [_bf(FX / tzc), jnp.zeros_like(itz), _bf(-FX * txc / (tzc * tzc))]
    J1 = [jnp.zeros_like(itz), _bf(FY / tzc), _bf(-FY * tyc / (tzc * tzc))]
    Tm0 = [J0[0] * vm[0][k] + J0[2] * vm[2][k] for k in range(3)]
    Tm1 = [J1[1] * vm[1][k] + J1[2] * vm[2][k] for k in range(3)]
    Tm0b = [_bf(t) for t in Tm0]
    Tm1b = [_bf(t) for t in Tm1]
    cov3b = [[_bf(cov3[a][b]) for b in range(3)] for a in range(3)]
    u0 = [_bf(sum(Tm0b[k] * cov3b[k][j] for k in range(3))) for j in range(3)]
    u1 = [_bf(sum(Tm1b[k] * cov3b[k][j] for k in range(3))) for j in range(3)]
    cov00 = sum(u0[j] * Tm0b[j] for j in range(3))
    cov01 = sum(u0[j] * Tm1b[j] for j in range(3))
    cov11 = sum(u1[j] * Tm1b[j] for j in range(3))

    a = cov00 + 0.3
    b = cov01
    c = cov11 + 0.3
    det = a * c - b * b
    valid = (det > 0.0) & (tz > 0.2)
    det_safe = jnp.where(valid, det, 1.0)
    conA = c / det_safe
    conB = -b / det_safe
    conC = a / det_safe
    px = ((ppx + 1.0) * W - 1.0) * 0.5
    py = ((ppy + 1.0) * H - 1.0) * 0.5
    mid = 0.5 * (a + c)
    l1 = mid + jnp.sqrt(jnp.maximum(mid * mid - det, 0.1))
    radii = jnp.where(valid, jnp.ceil(3.0 * jnp.sqrt(l1)), 0.0).astype(jnp.int32)
    opeff = jnp.where(valid, ot_ref[0:1, :], 0.0)

    zero = jnp.zeros_like(px)
    params_ref[...] = jnp.concatenate(
        [px, py, conA, conB + conB, conC, opeff, tz, zero],
        axis=0).astype(f32)
    colinv_ref[...] = jnp.concatenate(
        [ct_ref[0:1, :], ct_ref[1:2, :], ct_ref[2:3, :], itz], axis=0).astype(f32)
    radii_ref[...] = radii


def _raster_kernel(params_ref, colinv_ref, pg_ref, bg_ref, out_ref,
                   lT_ref, te_ref):
    f32 = jnp.float32
    jc = pl.program_id(1)
    pb = pl.program_id(0)

    @pl.when(jc == 0)
    def _init():
        lT_ref[...] = jnp.zeros_like(lT_ref)
        te_ref[...] = jnp.ones_like(te_ref)
        out_ref[...] = jnp.zeros_like(out_ref)

    alive = jnp.max(lT_ref[...]) >= LOG_EPS

    @pl.when(alive)
    def _compute():
        px = params_ref[0:1, :]
        py = params_ref[1:2, :]
        cA = params_ref[2:3, :]
        cB2 = params_ref[3:4, :]
        cC = params_ref[4:5, :]
        op = params_ref[5:6, :]
        pgx = pg_ref[:, 0:1]
        pgy = pg_ref[:, 1:2]
        dx = px - pgx
        dy = py - pgy
        power = -0.5 * ((cA * dx + cB2 * dy) * dx + (cC * dy) * dy)
        alpha = jnp.minimum(0.99, op * jnp.exp(jnp.minimum(power, 0.0)))
        alpha = jnp.where((power > 0.0) | (alpha < 1.0 / 255.0), 0.0, alpha)
        l1m = jnp.log(1.0 - alpha)
        rowi = jax.lax.broadcasted_iota(jnp.int32, (K_BLK, K_BLK), 0)
        coli = jax.lax.broadcasted_iota(jnp.int32, (K_BLK, K_BLK), 1)
        tri = (rowi <= coli).astype(f32)
        incl = jnp.dot(l1m, tri, preferred_element_type=f32)
        lT = lT_ref[...]
        te = te_ref[...]
        cb = _bf(colinv_ref[...])
        live = lT >= LOG_EPS
        lT_end = lT + incl[:, K_BLK - 1:K_BLK]
        no_cross = jnp.min(jnp.where(live, lT_end, 0.0)) >= LOG_EPS

        @pl.when(no_cross)
        def _fast():
            tel = jnp.where(live, te, 0.0)
            wgt = alpha * tel * jnp.exp(incl - l1m)
            out_ref[...] += jnp.dot(_bf(wgt), cb, preferred_element_type=f32)
            lT_ref[...] = lT_end
            te_ref[...] = jnp.where(
                live, te * jnp.exp(incl[:, K_BLK - 1:K_BLK]), te)

        @pl.when(jnp.logical_not(no_cross))
        def _slow():
            keep = ((lT + incl) >= LOG_EPS).astype(f32)
            mval = jnp.min(jnp.where(keep > 0.0, incl, 0.0), axis=1,
                           keepdims=True)
            incl_eff = jnp.maximum(incl, mval)
            excl_eff = incl_eff - l1m * keep
            wgt = alpha * keep * te * jnp.exp(excl_eff)
            out_ref[...] += jnp.dot(_bf(wgt), cb, preferred_element_type=f32)
            lT_ref[...] = lT_end
            te_ref[...] = te * jnp.exp(incl_eff[:, K_BLK - 1:K_BLK])

    @pl.when(jc == NC - 1)
    def _finish():
        out_ref[...] += te_ref[...] * bg_ref[...]


def _project(mt, st, qt, ot, ct, vm, pm):
    return pl.pallas_call(
        _project_kernel,
        out_shape=[
            jax.ShapeDtypeStruct((8, N), jnp.float32),
            jax.ShapeDtypeStruct((4, N), jnp.float32),
            jax.ShapeDtypeStruct((1, N), jnp.int32),
        ],
        in_specs=[
            pl.BlockSpec(memory_space=pltpu.VMEM),
            pl.BlockSpec(memory_space=pltpu.VMEM),
            pl.BlockSpec(memory_space=pltpu.VMEM),
            pl.BlockSpec(memory_space=pltpu.VMEM),
            pl.BlockSpec(memory_space=pltpu.VMEM),
            pl.BlockSpec(memory_space=pltpu.SMEM),
            pl.BlockSpec(memory_space=pltpu.SMEM),
        ],
    )(mt, st, qt, ot, ct, vm, pm)


def _rasterize(params_s, colinv_s, pg, bg4):
    return pl.pallas_call(
        _raster_kernel,
        grid=(PB, NC),
        in_specs=[
            pl.BlockSpec((8, K_BLK), lambda pb, jc: (0, jc)),
            pl.BlockSpec((K_BLK, 4), lambda pb, jc: (jc, 0)),
            pl.BlockSpec((P_BLK, 2), lambda pb, jc: (pb, 0)),
            pl.BlockSpec((1, 4), lambda pb, jc: (0, 0)),
        ],
        out_specs=pl.BlockSpec((P_BLK, 4), lambda pb, jc: (pb, 0)),
        out_shape=jax.ShapeDtypeStruct((HW, 4), jnp.float32),
        scratch_shapes=[
            pltpu.VMEM((P_BLK, 1), jnp.float32),
            pltpu.VMEM((P_BLK, 1), jnp.float32),
        ],
        compiler_params=pltpu.CompilerParams(
            dimension_semantics=("arbitrary", "arbitrary")),
    )(params_s, colinv_s, pg, bg4)


def kernel(means3D, means2D, opacities, colors_precomp, scales, rotations,
           viewmatrix, projmatrix, campos, bg):
    mt = means3D.T
    st = scales.T
    qt = rotations.T
    ot = opacities.T
    ct = colors_precomp.T
    vmb = viewmatrix.astype(jnp.bfloat16).astype(jnp.float32)
    pmb = projmatrix.astype(jnp.bfloat16).astype(jnp.float32)
    params, colinv_t, radii2 = _project(mt, st, qt, ot, ct, vmb, pmb)
    radii = radii2[0]
    order = jnp.argsort(params[6, :])
    params_s = params[:, order]
    colinv_s = colinv_t[:, order].T
    bg4 = jnp.concatenate([bg, jnp.zeros((1,), bg.dtype)])[None, :]
    pg = jnp.asarray(_PG)
    acc = _rasterize(params_s, colinv_s, pg, bg4)
    img = acc.reshape(H, W, 4)
    color = img[:, :, :3].transpose(2, 0, 1)
    invdepth = img[:, :, 3].reshape(1, H, W)
    return (color, radii, invdepth)


# speedup vs baseline: 2.0919x; 1.0080x over previous
import functools
import math

import jax
import jax.numpy as jnp
import numpy as np
from jax.experimental import pallas as pl
from jax.experimental.pallas import tpu as pltpu

N = 4096
H = 64
W = 64
HW = H * W
TANX = 0.5
TANY = 0.5
FX = W / (2.0 * TANX)
FY = H / (2.0 * TANY)
LIMX = 1.3 * TANX
LIMY = 1.3 * TANY
LOG_EPS = math.log(1e-4)

P_BLK = 4096
K_BLK = 512
NC = N // K_BLK
PB = HW // P_BLK

_yy, _xx = np.mgrid[0:H, 0:W]
_PG = np.stack([_xx.reshape(-1).astype(np.float32),
                _yy.reshape(-1).astype(np.float32)], axis=1)


def _bf(x):
    return x.astype(jnp.bfloat16).astype(jnp.float32)


def _project_kernel(mt_ref, st_ref, qt_ref, ot_ref, ct_ref, vm_ref, pm_ref,
                    params_ref, colinv_ref, radii_ref):
    f32 = jnp.float32
    mx = mt_ref[0:1, :]
    my = mt_ref[1:2, :]
    mz = mt_ref[2:3, :]
    s = [st_ref[i:i + 1, :] for i in range(3)]
    qr = qt_ref[0:1, :]
    qx = qt_ref[1:2, :]
    qy = qt_ref[2:3, :]
    qz = qt_ref[3:4, :]
    nrm = jnp.sqrt(qr * qr + qx * qx + qy * qy + qz * qz) + 1e-12
    r = qr / nrm
    x = qx / nrm
    y = qy / nrm
    z = qz / nrm
    R = [
        [1 - 2 * (y * y + z * z), 2 * (x * y - r * z), 2 * (x * z + r * y)],
        [2 * (x * y + r * z), 1 - 2 * (x * x + z * z), 2 * (y * z - r * x)],
        [2 * (x * z - r * y), 2 * (y * z + r * x), 1 - 2 * (x * x + y * y)],
    ]
    M = [[_bf(R[a][j] * s[j]) for j in range(3)] for a in range(3)]
    cov3 = [[sum(M[a][j] * M[b][j] for j in range(3)) for b in range(3)]
            for a in range(3)]

    vm = [[vm_ref[i, j] for j in range(4)] for i in range(4)]
    pm = [[pm_ref[i, j] for j in range(4)] for i in range(4)]
    mxb, myb, mzb = _bf(mx), _bf(my), _bf(mz)
    tx = vm[0][0] * mxb + vm[0][1] * myb + vm[0][2] * mzb + vm[0][3]
    ty = vm[1][0] * mxb + vm[1][1] * myb + vm[1][2] * mzb + vm[1][3]
    tz = vm[2][0] * mxb + vm[2][1] * myb + vm[2][2] * mzb + vm[2][3]
    ph0 = pm[0][0] * mxb + pm[0][1] * myb + pm[0][2] * mzb + pm[0][3]
    ph1 = pm[1][0] * mxb + pm[1][1] * myb + pm[1][2] * mzb + pm[1][3]
    ph3 = pm[3][0] * mxb + pm[3][1] * myb + pm[3][2] * mzb + pm[3][3]
    pw = 1.0 / (ph3 + 1e-7)
    ppx = ph0 * pw
    ppy = ph1 * pw

    tzc = jnp.where(jnp.abs(tz) < 1e-6, 1e-6, tz)
    txc = jnp.clip(tx / tzc, -LIMX, LIMX) * tzc
    tyc = jnp.clip(ty / tzc, -LIMY, LIMY) * tzc
    itz = 1.0 / tzc
    J0 = [_bf(FX / tzc), jnp.zeros_like(itz), _bf(-FX * txc / (tzc * tzc))]
    J1 = [jnp.zeros_like(itz), _bf(FY / tzc), _bf(-FY * tyc / (tzc * tzc))]
    Tm0 = [J0[0] * vm[0][k] + J0[2] * vm[2][k] for k in range(3)]
    Tm1 = [J1[1] * vm[1][k] + J1[2] * vm[2][k] for k in range(3)]
    Tm0b = [_bf(t) for t in Tm0]
    Tm1b = [_bf(t) for t in Tm1]
    cov3b = [[_bf(cov3[a][b]) for b in range(3)] for a in range(3)]
    u0 = [_bf(sum(Tm0b[k] * cov3b[k][j] for k in range(3))) for j in range(3)]
    u1 = [_bf(sum(Tm1b[k] * cov3b[k][j] for k in range(3))) for j in range(3)]
    cov00 = sum(u0[j] * Tm0b[j] for j in range(3))
    cov01 = sum(u0[j] * Tm1b[j] for j in range(3))
    cov11 = sum(u1[j] * Tm1b[j] for j in range(3))

    a = cov00 + 0.3
    b = cov01
    c = cov11 + 0.3
    det = a * c - b * b
    valid = (det > 0.0) & (tz > 0.2)
    det_safe = jnp.where(valid, det, 1.0)
    conA = c / det_safe
    conB = -b / det_safe
    conC = a / det_safe
    px = ((ppx + 1.0) * W - 1.0) * 0.5
    py = ((ppy + 1.0) * H - 1.0) * 0.5
    mid = 0.5 * (a + c)
    l1 = mid + jnp.sqrt(jnp.maximum(mid * mid - det, 0.1))
    radii = jnp.where(valid, jnp.ceil(3.0 * jnp.sqrt(l1)), 0.0).astype(jnp.int32)
    opeff = jnp.where(valid, ot_ref[0:1, :], 0.0)

    zero = jnp.zeros_like(px)
    params_ref[...] = jnp.concatenate(
        [px, py, conA, conB + conB, conC, opeff, tz, zero],
        axis=0).astype(f32)
    colinv_ref[...] = jnp.concatenate(
        [ct_ref[0:1, :], ct_ref[1:2, :], ct_ref[2:3, :], itz], axis=0).astype(f32)
    radii_ref[...] = radii


def _raster_kernel(params_ref, colinv_ref, pg_ref, bg_ref, out_ref,
                   lT_ref, te_ref):
    f32 = jnp.float32
    jc = pl.program_id(1)
    pb = pl.program_id(0)

    @pl.when(jc == 0)
    def _init():
        lT_ref[...] = jnp.zeros_like(lT_ref)
        te_ref[...] = jnp.ones_like(te_ref)
        out_ref[...] = jnp.zeros_like(out_ref)

    alive = jnp.max(lT_ref[...]) >= LOG_EPS

    @pl.when(alive)
    def _compute():
        px = params_ref[0:1, :]
        py = params_ref[1:2, :]
        cA = params_ref[2:3, :]
        cB2 = params_ref[3:4, :]
        cC = params_ref[4:5, :]
        op = params_ref[5:6, :]
        pgx = pg_ref[:, 0:1]
        pgy = pg_ref[:, 1:2]
        dx = px - pgx
        dy = py - pgy
        power = -0.5 * ((cA * dx + cB2 * dy) * dx + (cC * dy) * dy)
        alpha = jnp.minimum(0.99, op * jnp.exp(power))
        alpha = jnp.where((power > 0.0) | (alpha < 1.0 / 255.0), 0.0, alpha)
        l1m = jnp.log(1.0 - alpha)
        rowi = jax.lax.broadcasted_iota(jnp.int32, (K_BLK, K_BLK), 0)
        coli = jax.lax.broadcasted_iota(jnp.int32, (K_BLK, K_BLK), 1)
        tri = (rowi <= coli).astype(f32)
        incl = jnp.dot(l1m, tri, preferred_element_type=f32)
        lT = lT_ref[...]
        te = te_ref[...]
        cb = _bf(colinv_ref[...])
        live = lT >= LOG_EPS
        lT_end = lT + incl[:, K_BLK - 1:K_BLK]
        no_cross = jnp.min(jnp.where(live, lT_end, 0.0)) >= LOG_EPS

        @pl.when(no_cross)
        def _fast():
            tel = jnp.where(live, te, 0.0)
            wgt = alpha * tel * jnp.exp(incl - l1m)
            out_ref[...] += jnp.dot(_bf(wgt), cb, preferred_element_type=f32)
            lT_ref[...] = lT_end
            te_ref[...] = jnp.where(
                live, te * jnp.exp(incl[:, K_BLK - 1:K_BLK]), te)

        @pl.when(jnp.logical_not(no_cross))
        def _slow():
            keep = ((lT + incl) >= LOG_EPS).astype(f32)
            mval = jnp.min(jnp.where(keep > 0.0, incl, 0.0), axis=1,
                           keepdims=True)
            incl_eff = jnp.maximum(incl, mval)
            excl_eff = incl_eff - l1m * keep
            wgt = alpha * keep * te * jnp.exp(excl_eff)
            out_ref[...] += jnp.dot(_bf(wgt), cb, preferred_element_type=f32)
            lT_ref[...] = lT_end
            te_ref[...] = te * jnp.exp(incl_eff[:, K_BLK - 1:K_BLK])

    @pl.when(jc == NC - 1)
    def _finish():
        out_ref[...] += te_ref[...] * bg_ref[...]


def _project(mt, st, qt, ot, ct, vm, pm):
    return pl.pallas_call(
        _project_kernel,
        out_shape=[
            jax.ShapeDtypeStruct((8, N), jnp.float32),
            jax.ShapeDtypeStruct((4, N), jnp.float32),
            jax.ShapeDtypeStruct((1, N), jnp.int32),
        ],
        in_specs=[
            pl.BlockSpec(memory_space=pltpu.VMEM),
            pl.BlockSpec(memory_space=pltpu.VMEM),
            pl.BlockSpec(memory_space=pltpu.VMEM),
            pl.BlockSpec(memory_space=pltpu.VMEM),
            pl.BlockSpec(memory_space=pltpu.VMEM),
            pl.BlockSpec(memory_space=pltpu.SMEM),
            pl.BlockSpec(memory_space=pltpu.SMEM),
        ],
    )(mt, st, qt, ot, ct, vm, pm)


def _rasterize(params_s, colinv_s, pg, bg4):
    return pl.pallas_call(
        _raster_kernel,
        grid=(PB, NC),
        in_specs=[
            pl.BlockSpec((8, K_BLK), lambda pb, jc: (0, jc)),
            pl.BlockSpec((K_BLK, 4), lambda pb, jc: (jc, 0)),
            pl.BlockSpec((P_BLK, 2), lambda pb, jc: (pb, 0)),
            pl.BlockSpec((1, 4), lambda pb, jc: (0, 0)),
        ],
        out_specs=pl.BlockSpec((P_BLK, 4), lambda pb, jc: (pb, 0)),
        out_shape=jax.ShapeDtypeStruct((HW, 4), jnp.float32),
        scratch_shapes=[
            pltpu.VMEM((P_BLK, 1), jnp.float32),
            pltpu.VMEM((P_BLK, 1), jnp.float32),
        ],
        compiler_params=pltpu.CompilerParams(
            dimension_semantics=("arbitrary", "arbitrary")),
    )(params_s, colinv_s, pg, bg4)


def kernel(means3D, means2D, opacities, colors_precomp, scales, rotations,
           viewmatrix, projmatrix, campos, bg):
    mt = means3D.T
    st = scales.T
    qt = rotations.T
    ot = opacities.T
    ct = colors_precomp.T
    vmb = viewmatrix.astype(jnp.bfloat16).astype(jnp.float32)
    pmb = projmatrix.astype(jnp.bfloat16).astype(jnp.float32)
    params, colinv_t, radii2 = _project(mt, st, qt, ot, ct, vmb, pmb)
    radii = radii2[0]
    order = jnp.argsort(params[6, :])
    params_s = params[:, order]
    colinv_s = colinv_t[:, order].T
    bg4 = jnp.concatenate([bg, jnp.zeros((1,), bg.dtype)])[None, :]
    pg = jnp.asarray(_PG)
    acc = _rasterize(params_s, colinv_s, pg, bg4)
    img = acc.reshape(H, W, 4)
    color = img[:, :, :3].transpose(2, 0, 1)
    invdepth = img[:, :, 3].reshape(1, H, W)
    return (color, radii, invdepth)


# speedup vs baseline: 2.0943x; 1.0012x over previous
import math

import jax
import jax.numpy as jnp
import numpy as np
from jax.experimental import pallas as pl
from jax.experimental.pallas import tpu as pltpu

N = 4096
H = 64
W = 64
HW = H * W
TANX = 0.5
TANY = 0.5
FX = W / (2.0 * TANX)
FY = H / (2.0 * TANY)
LIMX = 1.3 * TANX
LIMY = 1.3 * TANY
LOG_EPS = math.log(1e-4)

P_BLK = 4096
K_BLK = 512
NC = N // K_BLK
PB = HW // P_BLK

_yy, _xx = np.mgrid[0:H, 0:W]
_PG = np.stack([_xx.reshape(-1).astype(np.float32),
                _yy.reshape(-1).astype(np.float32)], axis=1)


def _bf(x):
    return x.astype(jnp.bfloat16).astype(jnp.float32)


def _project_kernel(mt_ref, st_ref, qt_ref, ot_ref, ct_ref, vm_ref, pm_ref,
                    params_ref, colinv_ref, radii_ref):
    f32 = jnp.float32
    mx = mt_ref[0:1, :]
    my = mt_ref[1:2, :]
    mz = mt_ref[2:3, :]
    s = [st_ref[i:i + 1, :] for i in range(3)]
    qr = qt_ref[0:1, :]
    qx = qt_ref[1:2, :]
    qy = qt_ref[2:3, :]
    qz = qt_ref[3:4, :]
    nrm = jnp.sqrt(qr * qr + qx * qx + qy * qy + qz * qz) + 1e-12
    r = qr / nrm
    x = qx / nrm
    y = qy / nrm
    z = qz / nrm
    R = [
        [1 - 2 * (y * y + z * z), 2 * (x * y - r * z), 2 * (x * z + r * y)],
        [2 * (x * y + r * z), 1 - 2 * (x * x + z * z), 2 * (y * z - r * x)],
        [2 * (x * z - r * y), 2 * (y * z + r * x), 1 - 2 * (x * x + y * y)],
    ]
    M = [[_bf(R[a][j] * s[j]) for j in range(3)] for a in range(3)]
    cov3 = [[sum(M[a][j] * M[b][j] for j in range(3)) for b in range(3)]
            for a in range(3)]

    vm = [[vm_ref[i, j] for j in range(4)] for i in range(4)]
    pm = [[pm_ref[i, j] for j in range(4)] for i in range(4)]
    mxb, myb, mzb = _bf(mx), _bf(my), _bf(mz)
    tx = vm[0][0] * mxb + vm[0][1] * myb + vm[0][2] * mzb + vm[0][3]
    ty = vm[1][0] * mxb + vm[1][1] * myb + vm[1][2] * mzb + vm[1][3]
    tz = vm[2][0] * mxb + vm[2][1] * myb + vm[2][2] * mzb + vm[2][3]
    ph0 = pm[0][0] * mxb + pm[0][1] * myb + pm[0][2] * mzb + pm[0][3]
    ph1 = pm[1][0] * mxb + pm[1][1] * myb + pm[1][2] * mzb + pm[1][3]
    ph3 = pm[3][0] * mxb + pm[3][1] * myb + pm[3][2] * mzb + pm[3][3]
    pw = 1.0 / (ph3 + 1e-7)
    ppx = ph0 * pw
    ppy = ph1 * pw

    tzc = jnp.where(jnp.abs(tz) < 1e-6, 1e-6, tz)
    txc = jnp.clip(tx / tzc, -LIMX, LIMX) * tzc
    tyc = jnp.clip(ty / tzc, -LIMY, LIMY) * tzc
    itz = 1.0 / tzc
    J0 = [_bf(FX / tzc), jnp.zeros_like(itz), _bf(-FX * txc / (tzc * tzc))]
    J1 = [jnp.zeros_like(itz), _bf(FY / tzc), _bf(-FY * tyc / (tzc * tzc))]
    Tm0 = [J0[0] * vm[0][k] + J0[2] * vm[2][k] for k in range(3)]
    Tm1 = [J1[1] * vm[1][k] + J1[2] * vm[2][k] for k in range(3)]
    Tm0b = [_bf(t) for t in Tm0]
    Tm1b = [_bf(t) for t in Tm1]
    cov3b = [[_bf(cov3[a][b]) for b in range(3)] for a in range(3)]
    u0 = [_bf(sum(Tm0b[k] * cov3b[k][j] for k in range(3))) for j in range(3)]
    u1 = [_bf(sum(Tm1b[k] * cov3b[k][j] for k in range(3))) for j in range(3)]
    cov00 = sum(u0[j] * Tm0b[j] for j in range(3))
    cov01 = sum(u0[j] * Tm1b[j] for j in range(3))
    cov11 = sum(u1[j] * Tm1b[j] for j in range(3))

    a = cov00 + 0.3
    b = cov01
    c = cov11 + 0.3
    det = a * c - b * b
    valid = (det > 0.0) & (tz > 0.2)
    det_safe = jnp.where(valid, det, 1.0)
    conA = c / det_safe
    conB = -b / det_safe
    conC = a / det_safe
    px = ((ppx + 1.0) * W - 1.0) * 0.5
    py = ((ppy + 1.0) * H - 1.0) * 0.5
    mid = 0.5 * (a + c)
    l1 = mid + jnp.sqrt(jnp.maximum(mid * mid - det, 0.1))
    radii = jnp.where(valid, jnp.ceil(3.0 * jnp.sqrt(l1)), 0.0).astype(jnp.int32)
    opeff = jnp.where(valid, ot_ref[0:1, :], 0.0)

    zero = jnp.zeros_like(px)
    params_ref[...] = jnp.concatenate(
        [px, py, conA, conB + conB, conC, opeff, tz, zero],
        axis=0).astype(f32)
    colinv_ref[...] = jnp.concatenate(
        [ct_ref[0:1, :], ct_ref[1:2, :], ct_ref[2:3, :], itz], axis=0).astype(f32)
    radii_ref[...] = radii


def _raster_kernel(params_ref, colinv_ref, pg_ref, bg_ref, out_ref,
                   lT_ref, te_ref):
    f32 = jnp.float32
    jc = pl.program_id(1)

    @pl.when(jc == 0)
    def _init():
        lT_ref[...] = jnp.zeros_like(lT_ref)
        te_ref[...] = jnp.ones_like(te_ref)
        out_ref[...] = jnp.zeros_like(out_ref)

    alive = jnp.max(lT_ref[...]) >= LOG_EPS

    @pl.when(alive)
    def _compute():
        px = params_ref[0:1, :]
        py = params_ref[1:2, :]
        cA = params_ref[2:3, :]
        cB2 = params_ref[3:4, :]
        cC = params_ref[4:5, :]
        op = params_ref[5:6, :]
        pgx = pg_ref[:, 0:1]
        pgy = pg_ref[:, 1:2]
        dx = px - pgx
        dy = py - pgy
        power = -0.5 * ((cA * dx + cB2 * dy) * dx + (cC * dy) * dy)
        alpha = jnp.minimum(0.99, op * jnp.exp(power))
        alpha = jnp.where((power > 0.0) | (alpha < 1.0 / 255.0), 0.0, alpha)
        l1m = jnp.log(1.0 - alpha)
        rowi = jax.lax.broadcasted_iota(jnp.int32, (K_BLK, K_BLK), 0)
        coli = jax.lax.broadcasted_iota(jnp.int32, (K_BLK, K_BLK), 1)
        tri = (rowi <= coli).astype(f32)
        incl = jnp.dot(l1m, tri, preferred_element_type=f32)
        lT = lT_ref[...]
        te = te_ref[...]
        cb = _bf(colinv_ref[...])
        live = lT >= LOG_EPS
        lT_end = lT + incl[:, K_BLK - 1:K_BLK]
        no_cross = jnp.min(jnp.where(live, lT_end, 0.0)) >= LOG_EPS

        @pl.when(no_cross)
        def _fast():
            tel = jnp.where(live, te, 0.0)
            wgt = alpha * tel * jnp.exp(incl - l1m)
            out_ref[...] += jnp.dot(_bf(wgt), cb, preferred_element_type=f32)
            lT_ref[...] = lT_end
            te_ref[...] = jnp.where(
                live, te * jnp.exp(incl[:, K_BLK - 1:K_BLK]), te)

        @pl.when(jnp.logical_not(no_cross))
        def _slow():
            keep = ((lT + incl) >= LOG_EPS).astype(f32)
            mval = jnp.min(jnp.where(keep > 0.0, incl, 0.0), axis=1,
                           keepdims=True)
            incl_eff = jnp.maximum(incl, mval)
            excl_eff = incl_eff - l1m * keep
            wgt = alpha * keep * te * jnp.exp(excl_eff)
            out_ref[...] += jnp.dot(_bf(wgt), cb, preferred_element_type=f32)
            lT_ref[...] = lT_end
            te_ref[...] = te * jnp.exp(incl_eff[:, K_BLK - 1:K_BLK])

    @pl.when(jc == NC - 1)
    def _finish():
        out_ref[...] += te_ref[...] * bg_ref[...]


def _project(mt, st, qt, ot, ct, vm, pm):
    return pl.pallas_call(
        _project_kernel,
        out_shape=[
            jax.ShapeDtypeStruct((8, N), jnp.float32),
            jax.ShapeDtypeStruct((4, N), jnp.float32),
            jax.ShapeDtypeStruct((1, N), jnp.int32),
        ],
        in_specs=[
            pl.BlockSpec(memory_space=pltpu.VMEM),
            pl.BlockSpec(memory_space=pltpu.VMEM),
            pl.BlockSpec(memory_space=pltpu.VMEM),
            pl.BlockSpec(memory_space=pltpu.VMEM),
            pl.BlockSpec(memory_space=pltpu.VMEM),
            pl.BlockSpec(memory_space=pltpu.SMEM),
            pl.BlockSpec(memory_space=pltpu.SMEM),
        ],
    )(mt, st, qt, ot, ct, vm, pm)


def _rasterize(params_s, colinv_s, pg, bg4):
    return pl.pallas_call(
        _raster_kernel,
        grid=(PB, NC),
        in_specs=[
            pl.BlockSpec((8, K_BLK), lambda pb, jc: (0, jc)),
            pl.BlockSpec((K_BLK, 4), lambda pb, jc: (jc, 0)),
            pl.BlockSpec((P_BLK, 2), lambda pb, jc: (pb, 0)),
            pl.BlockSpec((1, 4), lambda pb, jc: (0, 0)),
        ],
        out_specs=pl.BlockSpec((P_BLK, 4), lambda pb, jc: (pb, 0)),
        out_shape=jax.ShapeDtypeStruct((HW, 4), jnp.float32),
        scratch_shapes=[
            pltpu.VMEM((P_BLK, 1), jnp.float32),
            pltpu.VMEM((P_BLK, 1), jnp.float32),
        ],
        compiler_params=pltpu.CompilerParams(
            dimension_semantics=("arbitrary", "arbitrary")),
    )(params_s, colinv_s, pg, bg4)


def kernel(means3D, means2D, opacities, colors_precomp, scales, rotations,
           viewmatrix, projmatrix, campos, bg):
    mt = means3D.T
    st = scales.T
    qt = rotations.T
    ot = opacities.T
    ct = colors_precomp.T
    vmb = viewmatrix.astype(jnp.bfloat16).astype(jnp.float32)
    pmb = projmatrix.astype(jnp.bfloat16).astype(jnp.float32)
    params, colinv_t, radii2 = _project(mt, st, qt, ot, ct, vmb, pmb)
    radii = radii2[0]
    order = jnp.argsort(params[6, :])
    params_s = params[:, order]
    colinv_s = colinv_t[:, order].T
    bg4 = jnp.concatenate([bg, jnp.zeros((1,), bg.dtype)])[None, :]
    pg = jnp.asarray(_PG)
    acc = _rasterize(params_s, colinv_s, pg, bg4)
    img = acc.reshape(H, W, 4)
    color = img[:, :, :3].transpose(2, 0, 1)
    invdepth = img[:, :, 3].reshape(1, H, W)
    return (color, radii, invdepth)
